# Initial kernel scaffold; baseline (speedup 1.0000x reference)
import jax
import jax.numpy as jnp
from jax.experimental import pallas as pl


def kernel(confidence, label, contrast_feature):
    raise NotImplementedError("write your pallas kernel here")


# speedup vs baseline: 2.4487x; 2.4487x over previous
import functools

import jax
import jax.numpy as jnp
from jax import lax
from jax.experimental import pallas as pl
from jax.experimental.pallas import tpu as pltpu
from jax.experimental.pallas import tpu_sc as plsc

NUM_CLASSES = 21
TOP_K = 64
B = 131072
D = 256
N_SEL = NUM_CLASSES * TOP_K
N_PAD = 2048


def _select_body(conf_ref, label_ref, out_ref):
    conf = conf_ref[...]
    label = label_ref[...]
    rows, cols = conf.shape
    neg_inf = jnp.finfo(jnp.float32).min
    idx2d = (lax.broadcasted_iota(jnp.int32, (rows, cols), 0) * cols
             + lax.broadcasted_iota(jnp.int32, (rows, cols), 1))
    out_iota = (lax.broadcasted_iota(jnp.int32, (16, 128), 0) * 128
                + lax.broadcasted_iota(jnp.int32, (16, 128), 1))
    acc0 = out_iota

    def body(t, carry):
        masked, acc = carry
        c = t // TOP_K
        fresh = jnp.where(label == c, conf, neg_inf)
        masked = jnp.where(t % TOP_K == 0, fresh, masked)
        m = jnp.max(masked)
        pick = jnp.min(jnp.where(masked == m, idx2d, jnp.int32(B)))
        masked = jnp.where(idx2d == pick, neg_inf, masked)
        acc = jnp.where(out_iota == t, pick, acc)
        return masked, acc

    _, acc = lax.fori_loop(0, N_SEL, body,
                           (jnp.full((rows, cols), neg_inf, jnp.float32), acc0))
    out_ref[...] = acc


def _tc_select(conf2, lab2, interpret=False):
    return pl.pallas_call(
        _select_body,
        out_shape=jax.ShapeDtypeStruct((16, 128), jnp.int32),
        interpret=interpret,
    )(conf2, lab2)


def _sc_gather(idx_flat, table):
    info = plsc.get_sparse_core_info()
    nc, ns = info.num_cores, info.num_subcores
    nw = nc * ns
    per_w = N_PAD // nw
    mesh = plsc.VectorSubcoreMesh(core_axis_name="c", subcore_axis_name="s")

    @functools.partial(
        pl.kernel,
        mesh=mesh,
        out_type=jax.ShapeDtypeStruct((N_PAD, D), jnp.float32),
        scratch_types=[
            pltpu.VMEM((per_w,), jnp.int32),
            pltpu.VMEM((per_w, D), jnp.float32),
            pltpu.SemaphoreType.DMA,
        ],
    )
    def gather_k(idx_hbm, table_hbm, out_hbm, idx_v, rows_v, sem):
        wid = lax.axis_index("s") * nc + lax.axis_index("c")
        base = wid * per_w
        pltpu.sync_copy(idx_hbm.at[pl.ds(base, per_w)], idx_v)
        pltpu.async_copy(table_hbm.at[idx_v], rows_v, sem).wait()
        pltpu.sync_copy(rows_v, out_hbm.at[pl.ds(base, per_w)])

    return gather_k(idx_flat, table)


def kernel(confidence, label, contrast_feature):
    conf2 = confidence.reshape(1024, 128)
    lab2 = label.reshape(1024, 128)
    order = _tc_select(conf2, lab2).reshape(N_PAD)
    rows = _sc_gather(order, contrast_feature)
    return rows[:N_SEL].reshape(NUM_CLASSES, TOP_K, D)
